# baseline (speedup 1.0000x reference)
import functools

import jax
import jax.numpy as jnp
from jax import lax
from jax.experimental import pallas as pl
from jax.experimental.pallas import tpu as pltpu
from jax.experimental.pallas import tpu_sc as plsc

N_NODES = 10000
N_EDGES = 320000
HID = 128
LEAK = 0.2

NW = 32
NPW = 320
NPAD = NW * NPW
ESC = N_EDGES // NW
ECH = 128
CAP = 10240
NREG = NW * NW + NW
NJ = HID // 16
NGRP = ESC // 256

_mesh = plsc.VectorSubcoreMesh(core_axis_name="c", subcore_axis_name="s")


def _leaky(v):
    return jnp.maximum(v, LEAK * v)


def _mm_body(x_ref, w_ref, b_ref, o_ref, *, act):
    acc = jnp.dot(x_ref[...], w_ref[...], preferred_element_type=jnp.float32)
    acc = acc + b_ref[...]
    o_ref[...] = _leaky(acc) if act else acc


def _matmul(x, w, b, act, m_block=2048):
    m, k = x.shape
    n = w.shape[1]
    return pl.pallas_call(
        functools.partial(_mm_body, act=act),
        grid=(m // m_block,),
        in_specs=[
            pl.BlockSpec((m_block, k), lambda i: (i, 0)),
            pl.BlockSpec((k, n), lambda i: (0, 0)),
            pl.BlockSpec((1, n), lambda i: (0, 0)),
        ],
        out_specs=pl.BlockSpec((m_block, n), lambda i: (i, 0)),
        out_shape=jax.ShapeDtypeStruct((m, n), jnp.float32),
    )(x, w, b.reshape(1, n))


def _mm2_body(x_ref, w_ref, b_ref, oa_ref, ob_ref):
    acc = jnp.dot(x_ref[...], w_ref[...], preferred_element_type=jnp.float32)
    acc = acc + b_ref[...]
    oa_ref[...] = acc[:, :HID]
    ob_ref[...] = acc[:, HID:]


def _matmul2(x, w, b, m_block=2048):
    m = x.shape[0]
    return pl.pallas_call(
        _mm2_body,
        grid=(m // m_block,),
        in_specs=[
            pl.BlockSpec((m_block, HID), lambda i: (i, 0)),
            pl.BlockSpec((HID, 2 * HID), lambda i: (0, 0)),
            pl.BlockSpec((1, 2 * HID), lambda i: (0, 0)),
        ],
        out_specs=[
            pl.BlockSpec((m_block, HID), lambda i: (i, 0)),
            pl.BlockSpec((m_block, HID), lambda i: (i, 0)),
        ],
        out_shape=[
            jax.ShapeDtypeStruct((m, HID), jnp.float32),
            jax.ShapeDtypeStruct((m, HID), jnp.float32),
        ],
    )(x, w, b.reshape(1, 2 * HID))


def _iota16():
    return lax.broadcasted_iota(jnp.int32, (16,), 0)


def _full16(val):
    return jnp.full((16,), val, jnp.int32)


@functools.partial(
    pl.kernel,
    out_type=[
        jax.ShapeDtypeStruct((NREG * CAP,), jnp.int32),
        jax.ShapeDtypeStruct((NREG * CAP,), jnp.int32),
        jax.ShapeDtypeStruct((NW * NW,), jnp.int32),
    ],
    mesh=_mesh,
    compiler_params=pltpu.CompilerParams(needs_layout_passes=False),
    scratch_types=[
        pltpu.VMEM((512,), jnp.int32),
        pltpu.VMEM((512,), jnp.int32),
        pltpu.VMEM((2, 128), jnp.int32),
        pltpu.VMEM((2, 128), jnp.int32),
        pltpu.VMEM((2, 128), jnp.int32),
        pltpu.VMEM((48,), jnp.int32),
        pltpu.SemaphoreType.DMA,
    ],
)
def _bucket_kernel(src_hbm, dst_hbm, bsrc_hbm, bldst_hbm, counts_hbm,
                   src_c, dst_c, idx_st, s_st, ld_st, cnt_v, sem):
    v = lax.axis_index("s") * 2 + lax.axis_index("c")
    iota = _iota16()
    zeros = jnp.zeros((16,), jnp.int32)

    for k in range(3):
        cnt_v[pl.ds(k * 16, 16)] = zeros

    def flush(h):
        pltpu.async_copy(s_st.at[h], bsrc_hbm.at[idx_st.at[h]], sem).wait()
        pltpu.async_copy(ld_st.at[h], bldst_hbm.at[idx_st.at[h]], sem).wait()

    def stage(h, sh, slot, s_vals, ld_vals):
        idx_st[h, pl.ds(sh * 16, 16)] = slot
        s_st[h, pl.ds(sh * 16, 16)] = s_vals
        ld_st[h, pl.ds(sh * 16, 16)] = ld_vals

    def process_real(el, h, sh):
        d = dst_c[pl.ds(el, 16)]
        wv = lax.div(d, _full16(NPW))
        kw, perm = plsc.sort_key_val(wv, el + iota)
        s_s = plsc.load_gather(src_c, [perm])
        d_s = plsc.load_gather(dst_c, [perm])
        ld_s = d_s - kw * NPW
        _, prev = plsc.sort_key_val(lax.rem(iota + 1, _full16(16)), kw)
        _, nxt = plsc.sort_key_val(lax.rem(iota + 15, _full16(16)), kw)
        start = (kw != prev) | (iota == 0)
        end = (kw != nxt) | (iota == 15)
        rank = iota - plsc.cummax(jnp.where(start, iota, zeros))
        cnt_cur = plsc.load_gather(cnt_v, [kw])
        pos = cnt_cur + rank
        plsc.store_scatter(cnt_v, [kw], pos + 1, mask=end)
        slot = (kw * NW + v) * CAP + pos
        stage(h, sh, slot, s_s, ld_s)

    def group_body(g, _):
        @pl.when(lax.rem(g, 2) == 0)
        def _():
            base = v * ESC + (g // 2) * 512
            pltpu.sync_copy(src_hbm.at[pl.ds(base, 512)], src_c)
            pltpu.sync_copy(dst_hbm.at[pl.ds(base, 512)], dst_c)
        for h in range(2):
            def step_body(sh, _, h=h):
                el = lax.rem(g, 2) * 256 + h * 128 + sh * 16
                process_real(el, h, sh)
                return 0
            lax.fori_loop(0, 8, step_body, 0)
            flush(h)
        return 0
    lax.fori_loop(0, NGRP, group_body, 0)

    dreg = (NW * NW + v) * CAP
    process_real(256, 0, 0)
    for sh in range(1, 8):
        stage(0, sh, dreg + (sh - 1) * 16 + iota, zeros, _full16(NPW))
    flush(0)
    for sh in range(8):
        stage(1, sh, dreg + 112 + sh * 16 + iota, zeros, _full16(NPW))
    flush(1)

    pltpu.sync_copy(cnt_v.at[pl.ds(0, NW)], counts_hbm.at[pl.ds(v * NW, NW)])

    for sh in range(8):
        for h in range(2):
            s_st[h, pl.ds(sh * 16, 16)] = zeros
            ld_st[h, pl.ds(sh * 16, 16)] = _full16(NPW)
    for w in range(NW):
        h = w % 2
        n_w = cnt_v[pl.ds(w, 16)][0]
        base = (w * NW + v) * CAP + n_w
        for k in range(8):
            idx_st[h, pl.ds(k * 16, 16)] = base + k * 16 + iota
        flush(h)


@functools.partial(
    pl.kernel,
    out_type=jax.ShapeDtypeStruct((NPAD, HID), jnp.float32),
    mesh=_mesh,
    compiler_params=pltpu.CompilerParams(needs_layout_passes=False),
    scratch_types=[
        pltpu.VMEM((NPW + 1, HID), jnp.float32),
        pltpu.VMEM((NPW + 1, HID), jnp.float32),
        pltpu.VMEM((ECH, HID), jnp.float32),
        pltpu.VMEM((ECH,), jnp.int32),
        pltpu.VMEM((ECH + 16,), jnp.int32),
        pltpu.VMEM((64, HID), jnp.float32),
        pltpu.VMEM((NW * NW + 16,), jnp.int32),
        pltpu.SemaphoreType.DMA,
    ],
)
def _edge_kernel(A_hbm, B_hbm, h_hbm, bsrc_hbm, bldst_hbm, counts_hbm,
                 hnew_hbm, b_v, acc, a_buf, sidx, sldst, hstage, cnts, sem):
    w = lax.axis_index("s") * 2 + lax.axis_index("c")
    pltpu.sync_copy(B_hbm.at[pl.ds(w * NPW, NPW)], b_v.at[pl.ds(0, NPW)])
    pltpu.sync_copy(counts_hbm, cnts.at[pl.ds(0, NW * NW)])

    neg_inf = jnp.full((16,), -jnp.inf, jnp.float32)

    def init_body(r, _):
        for j in range(NJ):
            acc[r, pl.ds(j * 16, 16)] = neg_inf
        return 0
    lax.fori_loop(0, NPW + 1, init_body, 0)

    def sub_body(v, _):
        n = cnts[pl.ds(v * NW + w, 16)][0]
        nch = (n + (ECH - 1)) // ECH
        base = (w * NW + v) * CAP

        def ch_body(c, _):
            slot0 = base + c * ECH
            pltpu.sync_copy(bsrc_hbm.at[pl.ds(slot0, ECH)], sidx)
            pltpu.sync_copy(bldst_hbm.at[pl.ds(slot0, ECH)],
                            sldst.at[pl.ds(0, ECH)])
            pltpu.async_copy(A_hbm.at[sidx], a_buf, sem).wait()

            def e_body(e, _):
                ld = sldst[pl.ds(e, 16)][0]
                for j in range(NJ):
                    dsj = pl.ds(j * 16, 16)
                    m = a_buf[e, dsj] + b_v[ld, dsj]
                    m = jnp.maximum(m, LEAK * m)
                    acc[ld, dsj] = jnp.maximum(acc[ld, dsj], m)
                return 0
            lax.fori_loop(0, ECH, e_body, 0)
            return 0
        lax.fori_loop(0, nch, ch_body, 0)
        return 0
    lax.fori_loop(0, NW, sub_body, 0)

    zerosf = jnp.zeros((16,), jnp.float32)
    for rc in range(NPW // 64):
        r0 = w * NPW + rc * 64
        pltpu.sync_copy(h_hbm.at[pl.ds(r0, 64)], hstage)

        def wb_body(r, _, rc=rc):
            for j in range(NJ):
                dsj = pl.ds(j * 16, 16)
                av = acc[rc * 64 + r, dsj]
                fin = jnp.where(av == -jnp.inf, zerosf, av)
                hstage[r, dsj] = hstage[r, dsj] + fin
            return 0
        lax.fori_loop(0, 64, wb_body, 0)
        pltpu.sync_copy(hstage, hnew_hbm.at[pl.ds(r0, 64)])


def kernel(x, edge_index, W_in, b_in, W_blocks, b_blocks, W_out, b_out):
    src = edge_index[0].astype(jnp.int32)
    dst = edge_index[1].astype(jnp.int32)
    pad = jnp.zeros((512,), jnp.int32)
    src_p = jnp.concatenate([src, pad])
    dst_p = jnp.concatenate([dst, pad])
    x_p = jnp.zeros((NPAD, x.shape[1]), jnp.float32).at[:N_NODES].set(x)

    h = _matmul(x_p, W_in, b_in, act=True)
    bsrc, bldst, counts = _bucket_kernel(src_p, dst_p)

    n_block = W_blocks.shape[0]
    for i in range(n_block):
        Wt = W_blocks[i, :HID, :]
        Wb = W_blocks[i, HID:, :]
        Wcat = jnp.concatenate([Wb, Wt - Wb], axis=1)
        bcat = jnp.concatenate([jnp.zeros_like(b_blocks[i]), b_blocks[i]])
        A, B = _matmul2(h, Wcat, bcat)
        h = _edge_kernel(A, B, h, bsrc, bldst, counts)

    out = _matmul(h, W_out, b_out, act=False)
    return out[:N_NODES]


# speedup vs baseline: 2.9396x; 2.9396x over previous
import functools

import jax
import jax.numpy as jnp
from jax import lax
from jax.experimental import pallas as pl
from jax.experimental.pallas import tpu as pltpu
from jax.experimental.pallas import tpu_sc as plsc

N_NODES = 10000
N_EDGES = 320000
HID = 128
LEAK = 0.2

NW = 32
NPW = 320
NPAD = NW * NPW
ESC = N_EDGES // NW
ECH = 128
CAPO = N_EDGES + ECH
DBASE = NW * CAPO
TAB = DBASE + NW * 256
NJ = HID // 16

_mesh = plsc.VectorSubcoreMesh(core_axis_name="c", subcore_axis_name="s")
_params = pltpu.CompilerParams(needs_layout_passes=False)


def _leaky(v):
    return jnp.maximum(v, LEAK * v)


def _mm_body(x_ref, w_ref, b_ref, o_ref, *, act):
    acc = jnp.dot(x_ref[...], w_ref[...], preferred_element_type=jnp.float32)
    acc = acc + b_ref[...]
    o_ref[...] = _leaky(acc) if act else acc


def _matmul(x, w, b, act, m_block=2048):
    m, k = x.shape
    n = w.shape[1]
    return pl.pallas_call(
        functools.partial(_mm_body, act=act),
        grid=(m // m_block,),
        in_specs=[
            pl.BlockSpec((m_block, k), lambda i: (i, 0)),
            pl.BlockSpec((k, n), lambda i: (0, 0)),
            pl.BlockSpec((1, n), lambda i: (0, 0)),
        ],
        out_specs=pl.BlockSpec((m_block, n), lambda i: (i, 0)),
        out_shape=jax.ShapeDtypeStruct((m, n), jnp.float32),
    )(x, w, b.reshape(1, n))


def _mm2_body(x_ref, w_ref, b_ref, oa_ref, ob_ref):
    acc = jnp.dot(x_ref[...], w_ref[...], preferred_element_type=jnp.float32)
    acc = acc + b_ref[...]
    oa_ref[...] = acc[:, :HID]
    ob_ref[...] = acc[:, HID:]


def _matmul2(x, w, b, m_block=2048):
    m = x.shape[0]
    return pl.pallas_call(
        _mm2_body,
        grid=(m // m_block,),
        in_specs=[
            pl.BlockSpec((m_block, HID), lambda i: (i, 0)),
            pl.BlockSpec((HID, 2 * HID), lambda i: (0, 0)),
            pl.BlockSpec((1, 2 * HID), lambda i: (0, 0)),
        ],
        out_specs=[
            pl.BlockSpec((m_block, HID), lambda i: (i, 0)),
            pl.BlockSpec((m_block, HID), lambda i: (i, 0)),
        ],
        out_shape=[
            jax.ShapeDtypeStruct((m, HID), jnp.float32),
            jax.ShapeDtypeStruct((m, HID), jnp.float32),
        ],
    )(x, w, b.reshape(1, 2 * HID))


def _iota16():
    return lax.broadcasted_iota(jnp.int32, (16,), 0)


def _full16(val):
    return jnp.full((16,), val, jnp.int32)


def _seg16(dst_vec, el_vec):
    iota = _iota16()
    wv = lax.div(dst_vec, _full16(NPW))
    kw, perm = plsc.sort_key_val(wv, el_vec)
    _, prev = plsc.sort_key_val(lax.rem(iota + 1, _full16(16)), kw)
    _, nxt = plsc.sort_key_val(lax.rem(iota + 15, _full16(16)), kw)
    start = (kw != prev) | (iota == 0)
    end = (kw != nxt) | (iota == 15)
    rank = iota - plsc.cummax(jnp.where(start, iota, jnp.zeros((16,), jnp.int32)))
    return kw, perm, rank, end


@functools.partial(
    pl.kernel,
    out_type=jax.ShapeDtypeStruct((NW * NW,), jnp.int32),
    mesh=_mesh,
    compiler_params=_params,
    scratch_types=[
        pltpu.VMEM((512,), jnp.int32),
        pltpu.VMEM((48,), jnp.int32),
    ],
)
def _count_kernel(dst_hbm, counts_hbm, dst_c, cnt_v):
    v = lax.axis_index("s") * 2 + lax.axis_index("c")
    zeros = jnp.zeros((16,), jnp.int32)
    for k in range(3):
        cnt_v[pl.ds(k * 16, 16)] = zeros

    def chunk_body(t, _):
        pltpu.sync_copy(dst_hbm.at[pl.ds(v * ESC + t * 512, 512)], dst_c)

        def step_body(s, _):
            @pl.when(t * 512 + s * 16 < ESC)
            def _():
                d = dst_c[pl.ds(s * 16, 16)]
                kw, _, rank, end = _seg16(d, _iota16())
                cnt_cur = plsc.load_gather(cnt_v, [kw])
                plsc.store_scatter(cnt_v, [kw], cnt_cur + rank + 1, mask=end)
            return 0
        lax.fori_loop(0, 32, step_body, 0)
        return 0
    lax.fori_loop(0, (ESC + 511) // 512, chunk_body, 0)
    pltpu.sync_copy(cnt_v.at[pl.ds(0, NW)], counts_hbm.at[pl.ds(v * NW, NW)])


@functools.partial(
    pl.kernel,
    out_type=[
        jax.ShapeDtypeStruct((TAB,), jnp.int32),
        jax.ShapeDtypeStruct((TAB,), jnp.int32),
    ],
    mesh=_mesh,
    compiler_params=_params,
    scratch_types=[
        pltpu.VMEM((1024,), jnp.int32),
        pltpu.VMEM((1024,), jnp.int32),
        pltpu.VMEM((2, 4, 128), jnp.int32),
        pltpu.VMEM((2, 4, 128), jnp.int32),
        pltpu.VMEM((2, 4, 128), jnp.int32),
        pltpu.VMEM((48,), jnp.int32),
        pltpu.VMEM((NW * NW + 16,), jnp.int32),
        pltpu.SemaphoreType.DMA,
    ],
)
def _scatter_kernel(src_hbm, dst_hbm, counts_hbm, bsrc_hbm, bldst_hbm,
                    src_c, dst_c, idx_st, s_st, ld_st, cnt_v, cnts, sem):
    v = lax.axis_index("s") * 2 + lax.axis_index("c")
    iota = _iota16()
    zeros = jnp.zeros((16,), jnp.int32)
    pltpu.sync_copy(counts_hbm, cnts.at[pl.ds(0, NW * NW)])

    for wg in range(2):
        wbase = (wg * 16 + iota) * CAPO

        def pref_body(vp, a, wg=wg):
            return a + cnts[pl.ds(vp * NW + wg * 16, 16)]
        cnt_v[pl.ds(wg * 16, 16)] = wbase + lax.fori_loop(
            0, v, pref_body, zeros)

    def fire(h, k):
        pltpu.async_copy(s_st.at[h, k], bsrc_hbm.at[idx_st.at[h, k]], sem)
        pltpu.async_copy(ld_st.at[h, k], bldst_hbm.at[idx_st.at[h, k]], sem)

    def drain(h, k):
        pltpu.make_async_copy(
            s_st.at[h, k], bsrc_hbm.at[idx_st.at[h, k]], sem).wait()
        pltpu.make_async_copy(
            ld_st.at[h, k], bldst_hbm.at[idx_st.at[h, k]], sem).wait()

    def step(g, h, sh):
        el = h * 512 + sh * 16
        e_g = g * 1024 + el
        k = sh // 8
        off = lax.rem(sh, 8) * 16

        @pl.when(e_g < ESC)
        def _():
            d = dst_c[pl.ds(el, 16)]
            kw, perm, rank, end = _seg16(d, el + iota)
            s_s = plsc.load_gather(src_c, [perm])
            d_s = plsc.load_gather(dst_c, [perm])
            ld_s = d_s - kw * NPW
            cnt_cur = plsc.load_gather(cnt_v, [kw])
            pos = cnt_cur + rank
            plsc.store_scatter(cnt_v, [kw], pos + 1, mask=end)
            idx_st[h, k, pl.ds(off, 16)] = pos
            s_st[h, k, pl.ds(off, 16)] = s_s
            ld_st[h, k, pl.ds(off, 16)] = ld_s

        @pl.when(e_g >= ESC)
        def _():
            idx_st[h, k, pl.ds(off, 16)] = DBASE + v * 256 + (e_g - ESC) + iota
            s_st[h, k, pl.ds(off, 16)] = zeros
            ld_st[h, k, pl.ds(off, 16)] = _full16(NPW)

    def group_body(g, _):
        pltpu.sync_copy(src_hbm.at[pl.ds(v * ESC + g * 1024, 1024)], src_c)
        pltpu.sync_copy(dst_hbm.at[pl.ds(v * ESC + g * 1024, 1024)], dst_c)
        for h in range(2):
            @pl.when(g > 0)
            def _(h=h):
                for k in range(4):
                    drain(h, k)

            def half_body(sh, _, h=h):
                step(g, h, sh)
                return 0
            lax.fori_loop(0, 32, half_body, 0)
            for k in range(4):
                fire(h, k)
        return 0
    lax.fori_loop(0, (ESC + 1023) // 1024, group_body, 0)
    for h in range(2):
        for k in range(4):
            drain(h, k)

    def tot_body(vp, a):
        return a + cnts[pl.ds(vp * NW + v, 16)][0]
    tot = v * CAPO + lax.fori_loop(0, NW, tot_body, 0)
    for k in range(8):
        idx_st[0, 0, pl.ds(k * 16, 16)] = tot + k * 16 + iota
        s_st[0, 0, pl.ds(k * 16, 16)] = zeros
        ld_st[0, 0, pl.ds(k * 16, 16)] = _full16(NPW)
    fire(0, 0)
    drain(0, 0)


@functools.partial(
    pl.kernel,
    out_type=jax.ShapeDtypeStruct((NPAD, HID), jnp.float32),
    mesh=_mesh,
    compiler_params=_params,
    scratch_types=[
        pltpu.VMEM((NPW + 1, HID), jnp.float32),
        pltpu.VMEM((NPW + 1, HID), jnp.float32),
        pltpu.VMEM((2, ECH, HID), jnp.float32),
        pltpu.VMEM((16 * ECH,), jnp.int32),
        pltpu.VMEM((16 * ECH + 16,), jnp.int32),
        pltpu.VMEM((64, HID), jnp.float32),
        pltpu.VMEM((NW * NW + 16,), jnp.int32),
        pltpu.SemaphoreType.DMA,
    ],
)
def _edge_kernel(A_hbm, B_hbm, h_hbm, bsrc_hbm, bldst_hbm, counts_hbm,
                 hnew_hbm, b_v, acc, a_buf, sidx, sldst, hstage, cnts, sem):
    w = lax.axis_index("s") * 2 + lax.axis_index("c")
    iota = _iota16()
    pltpu.sync_copy(B_hbm.at[pl.ds(w * NPW, NPW)], b_v.at[pl.ds(0, NPW)])
    pltpu.sync_copy(counts_hbm, cnts.at[pl.ds(0, NW * NW)])

    neg_inf = jnp.full((16,), -jnp.inf, jnp.float32)

    def init_body(r, _):
        for j in range(NJ):
            acc[r, pl.ds(j * 16, 16)] = neg_inf
        return 0
    lax.fori_loop(0, NPW + 1, init_body, 0)

    tot = (plsc.load_gather(cnts, [iota * NW + w])
           + plsc.load_gather(cnts, [(iota + 16) * NW + w]))
    n = jnp.sum(tot)
    nch = (n + (ECH - 1)) // ECH
    base = w * CAPO

    def fetch_slab(sl):
        p0 = lax.rem(sl, 2) * 1024
        pltpu.sync_copy(bsrc_hbm.at[pl.ds(base + sl * 1024, 1024)],
                        sidx.at[pl.ds(p0, 1024)])
        pltpu.sync_copy(bldst_hbm.at[pl.ds(base + sl * 1024, 1024)],
                        sldst.at[pl.ds(p0, 1024)])

    def gather_start(c):
        pltpu.async_copy(A_hbm.at[sidx.at[pl.ds(lax.rem(c, 16) * ECH, ECH)]],
                         a_buf.at[lax.rem(c, 2)], sem)

    def gather_wait(c):
        pltpu.make_async_copy(
            A_hbm.at[sidx.at[pl.ds(lax.rem(c, 16) * ECH, ECH)]],
            a_buf.at[lax.rem(c, 2)], sem).wait()

    @pl.when(nch > 0)
    def _():
        fetch_slab(0)
        gather_start(0)

        def ch_body(c, _):
            gather_wait(c)

            @pl.when((lax.rem(c, 8) == 7) & (c + 1 < nch))
            def _():
                fetch_slab((c + 1) // 8)

            @pl.when(c + 1 < nch)
            def _():
                gather_start(c + 1)

            buf = lax.rem(c, 2)
            r0 = lax.rem(c, 16) * ECH

            def e_body(e, _):
                ld = sldst[pl.ds(r0 + e, 16)][0]
                for j in range(NJ):
                    dsj = pl.ds(j * 16, 16)
                    m = a_buf[buf, e, dsj] + b_v[ld, dsj]
                    m = jnp.maximum(m, LEAK * m)
                    acc[ld, dsj] = jnp.maximum(acc[ld, dsj], m)
                return 0
            lax.fori_loop(0, ECH, e_body, 0, unroll=2)
            return 0
        lax.fori_loop(0, nch, ch_body, 0)

    zerosf = jnp.zeros((16,), jnp.float32)
    for rc in range(NPW // 64):
        r0 = w * NPW + rc * 64
        pltpu.sync_copy(h_hbm.at[pl.ds(r0, 64)], hstage)

        def wb_body(r, _, rc=rc):
            for j in range(NJ):
                dsj = pl.ds(j * 16, 16)
                av = acc[rc * 64 + r, dsj]
                fin = jnp.where(av == -jnp.inf, zerosf, av)
                hstage[r, dsj] = hstage[r, dsj] + fin
            return 0
        lax.fori_loop(0, 64, wb_body, 0)
        pltpu.sync_copy(hstage, hnew_hbm.at[pl.ds(r0, 64)])


def kernel(x, edge_index, W_in, b_in, W_blocks, b_blocks, W_out, b_out):
    src = edge_index[0].astype(jnp.int32)
    dst = edge_index[1].astype(jnp.int32)
    pad = jnp.zeros((1024,), jnp.int32)
    src_p = jnp.concatenate([src, pad])
    dst_p = jnp.concatenate([dst, pad])
    x_p = jnp.zeros((NPAD, x.shape[1]), jnp.float32).at[:N_NODES].set(x)

    h = _matmul(x_p, W_in, b_in, act=True)
    counts = _count_kernel(dst_p)
    bsrc, bldst = _scatter_kernel(src_p, dst_p, counts)

    n_block = W_blocks.shape[0]
    for i in range(n_block):
        Wt = W_blocks[i, :HID, :]
        Wb = W_blocks[i, HID:, :]
        Wcat = jnp.concatenate([Wb, Wt - Wb], axis=1)
        bcat = jnp.concatenate([jnp.zeros_like(b_blocks[i]), b_blocks[i]])
        A, B = _matmul2(h, Wcat, bcat)
        h = _edge_kernel(A, B, h, bsrc, bldst, counts)

    out = _matmul(h, W_out, b_out, act=False)
    return out[:N_NODES]


# speedup vs baseline: 2.9859x; 1.0157x over previous
import functools

import jax
import jax.numpy as jnp
from jax import lax
from jax.experimental import pallas as pl
from jax.experimental.pallas import tpu as pltpu
from jax.experimental.pallas import tpu_sc as plsc

N_NODES = 10000
N_EDGES = 320000
HID = 128
LEAK = 0.2

NW = 32
NPW = 320
NPAD = NW * NPW
ESC = N_EDGES // NW
ECH = 128
CAPO = N_EDGES + ECH
DBASE = NW * CAPO
TAB = DBASE + NW * 256
NJ = HID // 16

_mesh = plsc.VectorSubcoreMesh(core_axis_name="c", subcore_axis_name="s")
_params = pltpu.CompilerParams(needs_layout_passes=False)


def _leaky(v):
    return jnp.maximum(v, LEAK * v)


def _mm_body(x_ref, w_ref, b_ref, o_ref, *, act):
    acc = jnp.dot(x_ref[...], w_ref[...], preferred_element_type=jnp.float32)
    acc = acc + b_ref[...]
    o_ref[...] = _leaky(acc) if act else acc


def _matmul(x, w, b, act, m_block=2048):
    m, k = x.shape
    n = w.shape[1]
    return pl.pallas_call(
        functools.partial(_mm_body, act=act),
        grid=(m // m_block,),
        in_specs=[
            pl.BlockSpec((m_block, k), lambda i: (i, 0)),
            pl.BlockSpec((k, n), lambda i: (0, 0)),
            pl.BlockSpec((1, n), lambda i: (0, 0)),
        ],
        out_specs=pl.BlockSpec((m_block, n), lambda i: (i, 0)),
        out_shape=jax.ShapeDtypeStruct((m, n), jnp.float32),
    )(x, w, b.reshape(1, n))


def _mm2_body(x_ref, w_ref, b_ref, oa_ref, ob_ref):
    acc = jnp.dot(x_ref[...], w_ref[...], preferred_element_type=jnp.float32)
    acc = acc + b_ref[...]
    oa_ref[...] = acc[:, :HID]
    ob_ref[...] = acc[:, HID:]


def _matmul2(x, w, b, m_block=2048):
    m = x.shape[0]
    return pl.pallas_call(
        _mm2_body,
        grid=(m // m_block,),
        in_specs=[
            pl.BlockSpec((m_block, HID), lambda i: (i, 0)),
            pl.BlockSpec((HID, 2 * HID), lambda i: (0, 0)),
            pl.BlockSpec((1, 2 * HID), lambda i: (0, 0)),
        ],
        out_specs=[
            pl.BlockSpec((m_block, HID), lambda i: (i, 0)),
            pl.BlockSpec((m_block, HID), lambda i: (i, 0)),
        ],
        out_shape=[
            jax.ShapeDtypeStruct((m, HID), jnp.float32),
            jax.ShapeDtypeStruct((m, HID), jnp.float32),
        ],
    )(x, w, b.reshape(1, 2 * HID))


def _iota16():
    return lax.broadcasted_iota(jnp.int32, (16,), 0)


def _full16(val):
    return jnp.full((16,), val, jnp.int32)


def _seg16(dst_vec, el_vec):
    iota = _iota16()
    wv = lax.div(dst_vec, _full16(NPW))
    kw, perm = plsc.sort_key_val(wv, el_vec)
    _, prev = plsc.sort_key_val(lax.rem(iota + 1, _full16(16)), kw)
    _, nxt = plsc.sort_key_val(lax.rem(iota + 15, _full16(16)), kw)
    start = (kw != prev) | (iota == 0)
    end = (kw != nxt) | (iota == 15)
    rank = iota - plsc.cummax(jnp.where(start, iota, jnp.zeros((16,), jnp.int32)))
    return kw, perm, rank, end


@functools.partial(
    pl.kernel,
    out_type=jax.ShapeDtypeStruct((NW * NW,), jnp.int32),
    mesh=_mesh,
    compiler_params=_params,
    scratch_types=[
        pltpu.VMEM((2 * 512,), jnp.int32),
        pltpu.VMEM((48,), jnp.int32),
        pltpu.SemaphoreType.DMA,
    ],
)
def _count_kernel(dst_hbm, counts_hbm, dst_c, cnt_v, sem):
    v = lax.axis_index("s") * 2 + lax.axis_index("c")
    zeros = jnp.zeros((16,), jnp.int32)
    for k in range(3):
        cnt_v[pl.ds(k * 16, 16)] = zeros
    nchunk = (ESC + 511) // 512

    def fetch(t):
        pltpu.async_copy(dst_hbm.at[pl.ds(v * ESC + t * 512, 512)],
                         dst_c.at[pl.ds(lax.rem(t, 2) * 512, 512)], sem)

    def fetch_wait(t):
        pltpu.make_async_copy(
            dst_hbm.at[pl.ds(v * ESC + t * 512, 512)],
            dst_c.at[pl.ds(lax.rem(t, 2) * 512, 512)], sem).wait()

    fetch(0)

    def chunk_body(t, _):
        fetch_wait(t)

        @pl.when(t + 1 < nchunk)
        def _():
            fetch(t + 1)
        p0 = lax.rem(t, 2) * 512

        def step_body(s, _):
            @pl.when(t * 512 + s * 16 < ESC)
            def _():
                d = dst_c[pl.ds(p0 + s * 16, 16)]
                kw, _, rank, end = _seg16(d, _iota16())
                cnt_cur = plsc.load_gather(cnt_v, [kw])
                plsc.store_scatter(cnt_v, [kw], cnt_cur + rank + 1, mask=end)
            return 0
        lax.fori_loop(0, 32, step_body, 0)
        return 0
    lax.fori_loop(0, nchunk, chunk_body, 0)
    pltpu.sync_copy(cnt_v.at[pl.ds(0, NW)], counts_hbm.at[pl.ds(v * NW, NW)])


@functools.partial(
    pl.kernel,
    out_type=[
        jax.ShapeDtypeStruct((TAB,), jnp.int32),
        jax.ShapeDtypeStruct((TAB,), jnp.int32),
    ],
    mesh=_mesh,
    compiler_params=_params,
    scratch_types=[
        pltpu.VMEM((2 * 1024,), jnp.int32),
        pltpu.VMEM((2 * 1024,), jnp.int32),
        pltpu.VMEM((2, 4, 128), jnp.int32),
        pltpu.VMEM((2, 4, 128), jnp.int32),
        pltpu.VMEM((2, 4, 128), jnp.int32),
        pltpu.VMEM((48,), jnp.int32),
        pltpu.VMEM((NW * NW + 16,), jnp.int32),
        pltpu.SemaphoreType.DMA,
        pltpu.SemaphoreType.DMA,
    ],
)
def _scatter_kernel(src_hbm, dst_hbm, counts_hbm, bsrc_hbm, bldst_hbm,
                    src_c, dst_c, idx_st, s_st, ld_st, cnt_v, cnts, sem,
                    sem_f):
    v = lax.axis_index("s") * 2 + lax.axis_index("c")
    iota = _iota16()
    zeros = jnp.zeros((16,), jnp.int32)
    pltpu.sync_copy(counts_hbm, cnts.at[pl.ds(0, NW * NW)])

    for wg in range(2):
        wbase = (wg * 16 + iota) * CAPO

        def pref_body(vp, a, wg=wg):
            return a + cnts[pl.ds(vp * NW + wg * 16, 16)]
        cnt_v[pl.ds(wg * 16, 16)] = wbase + lax.fori_loop(
            0, v, pref_body, zeros)

    def fire(h, k):
        pltpu.async_copy(s_st.at[h, k], bsrc_hbm.at[idx_st.at[h, k]], sem)
        pltpu.async_copy(ld_st.at[h, k], bldst_hbm.at[idx_st.at[h, k]], sem)

    def drain(h, k):
        pltpu.make_async_copy(
            s_st.at[h, k], bsrc_hbm.at[idx_st.at[h, k]], sem).wait()
        pltpu.make_async_copy(
            ld_st.at[h, k], bldst_hbm.at[idx_st.at[h, k]], sem).wait()

    def step(g, h, sh):
        p0 = lax.rem(g, 2) * 1024
        el = p0 + h * 512 + sh * 16
        e_g = g * 1024 + h * 512 + sh * 16
        k = sh // 8
        off = lax.rem(sh, 8) * 16

        @pl.when(e_g < ESC)
        def _():
            d = dst_c[pl.ds(el, 16)]
            kw, perm, rank, end = _seg16(d, el - p0 + iota)
            s_s = plsc.load_gather(src_c, [p0 + perm])
            d_s = plsc.load_gather(dst_c, [p0 + perm])
            ld_s = d_s - kw * NPW
            cnt_cur = plsc.load_gather(cnt_v, [kw])
            pos = cnt_cur + rank
            plsc.store_scatter(cnt_v, [kw], pos + 1, mask=end)
            idx_st[h, k, pl.ds(off, 16)] = pos
            s_st[h, k, pl.ds(off, 16)] = s_s
            ld_st[h, k, pl.ds(off, 16)] = ld_s

        @pl.when(e_g >= ESC)
        def _():
            idx_st[h, k, pl.ds(off, 16)] = DBASE + v * 256 + (e_g - ESC) + iota
            s_st[h, k, pl.ds(off, 16)] = zeros
            ld_st[h, k, pl.ds(off, 16)] = _full16(NPW)

    ngrp = (ESC + 1023) // 1024

    def fetch(g):
        p0 = lax.rem(g, 2) * 1024
        pltpu.async_copy(src_hbm.at[pl.ds(v * ESC + g * 1024, 1024)],
                         src_c.at[pl.ds(p0, 1024)], sem_f)
        pltpu.async_copy(dst_hbm.at[pl.ds(v * ESC + g * 1024, 1024)],
                         dst_c.at[pl.ds(p0, 1024)], sem_f)

    def fetch_wait(g):
        p0 = lax.rem(g, 2) * 1024
        pltpu.make_async_copy(src_hbm.at[pl.ds(v * ESC + g * 1024, 1024)],
                              src_c.at[pl.ds(p0, 1024)], sem_f).wait()
        pltpu.make_async_copy(dst_hbm.at[pl.ds(v * ESC + g * 1024, 1024)],
                              dst_c.at[pl.ds(p0, 1024)], sem_f).wait()

    fetch(0)

    def group_body(g, _):
        fetch_wait(g)

        @pl.when(g + 1 < ngrp)
        def _():
            fetch(g + 1)
        for h in range(2):
            @pl.when(g > 0)
            def _(h=h):
                for k in range(4):
                    drain(h, k)

            def half_body(sh, _, h=h):
                step(g, h, sh)
                return 0
            lax.fori_loop(0, 32, half_body, 0)
            for k in range(4):
                fire(h, k)
        return 0
    lax.fori_loop(0, (ESC + 1023) // 1024, group_body, 0)
    for h in range(2):
        for k in range(4):
            drain(h, k)

    def tot_body(vp, a):
        return a + cnts[pl.ds(vp * NW + v, 16)][0]
    tot = v * CAPO + lax.fori_loop(0, NW, tot_body, 0)
    for k in range(8):
        idx_st[0, 0, pl.ds(k * 16, 16)] = tot + k * 16 + iota
        s_st[0, 0, pl.ds(k * 16, 16)] = zeros
        ld_st[0, 0, pl.ds(k * 16, 16)] = _full16(NPW)
    fire(0, 0)
    drain(0, 0)


@functools.partial(
    pl.kernel,
    out_type=jax.ShapeDtypeStruct((NPAD, HID), jnp.float32),
    mesh=_mesh,
    compiler_params=_params,
    scratch_types=[
        pltpu.VMEM((NPW + 1, HID), jnp.float32),
        pltpu.VMEM((NPW + 1, HID), jnp.float32),
        pltpu.VMEM((2, ECH, HID), jnp.float32),
        pltpu.VMEM((16 * ECH,), jnp.int32),
        pltpu.VMEM((16 * ECH + 16,), jnp.int32),
        pltpu.VMEM((2, 32, HID), jnp.float32),
        pltpu.VMEM((NW * NW + 16,), jnp.int32),
        pltpu.SemaphoreType.DMA,
        pltpu.SemaphoreType.DMA,
        pltpu.SemaphoreType.DMA,
        pltpu.SemaphoreType.DMA,
    ],
)
def _edge_kernel(A_hbm, B_hbm, h_hbm, bsrc_hbm, bldst_hbm, counts_hbm,
                 hnew_hbm, b_v, acc, a_buf, sidx, sldst, hstage, cnts, sem,
                 sem_s, sem_wl, sem_ws):
    w = lax.axis_index("s") * 2 + lax.axis_index("c")
    iota = _iota16()
    pltpu.async_copy(B_hbm.at[pl.ds(w * NPW, NPW)], b_v.at[pl.ds(0, NPW)],
                     sem_wl)
    pltpu.async_copy(counts_hbm, cnts.at[pl.ds(0, NW * NW)], sem_ws)

    neg_inf = jnp.full((16,), -jnp.inf, jnp.float32)

    def init_body(r, _):
        for j in range(NJ):
            acc[r, pl.ds(j * 16, 16)] = neg_inf
        return 0
    lax.fori_loop(0, NPW + 1, init_body, 0)
    pltpu.make_async_copy(B_hbm.at[pl.ds(w * NPW, NPW)],
                          b_v.at[pl.ds(0, NPW)], sem_wl).wait()
    pltpu.make_async_copy(counts_hbm, cnts.at[pl.ds(0, NW * NW)],
                          sem_ws).wait()

    tot = (plsc.load_gather(cnts, [iota * NW + w])
           + plsc.load_gather(cnts, [(iota + 16) * NW + w]))
    n = jnp.sum(tot)
    nch = (n + (ECH - 1)) // ECH
    base = w * CAPO

    def fetch_slab(sl):
        p0 = lax.rem(sl, 2) * 1024
        pltpu.async_copy(bsrc_hbm.at[pl.ds(base + sl * 1024, 1024)],
                         sidx.at[pl.ds(p0, 1024)], sem_s)
        pltpu.async_copy(bldst_hbm.at[pl.ds(base + sl * 1024, 1024)],
                         sldst.at[pl.ds(p0, 1024)], sem_s)

    def fetch_slab_wait(sl):
        p0 = lax.rem(sl, 2) * 1024
        pltpu.make_async_copy(bsrc_hbm.at[pl.ds(base + sl * 1024, 1024)],
                              sidx.at[pl.ds(p0, 1024)], sem_s).wait()
        pltpu.make_async_copy(bldst_hbm.at[pl.ds(base + sl * 1024, 1024)],
                              sldst.at[pl.ds(p0, 1024)], sem_s).wait()

    def gather_start(c):
        pltpu.async_copy(A_hbm.at[sidx.at[pl.ds(lax.rem(c, 16) * ECH, ECH)]],
                         a_buf.at[lax.rem(c, 2)], sem)

    def gather_wait(c):
        pltpu.make_async_copy(
            A_hbm.at[sidx.at[pl.ds(lax.rem(c, 16) * ECH, ECH)]],
            a_buf.at[lax.rem(c, 2)], sem).wait()

    @pl.when(nch > 0)
    def _():
        fetch_slab(0)
        fetch_slab_wait(0)
        gather_start(0)

        def ch_body(c, _):
            gather_wait(c)

            @pl.when((lax.rem(c, 8) == 0) & (c + 8 < nch))
            def _():
                fetch_slab(c // 8 + 1)

            @pl.when((lax.rem(c, 8) == 7) & (c + 1 < nch))
            def _():
                fetch_slab_wait((c + 1) // 8)

            @pl.when(c + 1 < nch)
            def _():
                gather_start(c + 1)

            buf = lax.rem(c, 2)
            r0 = lax.rem(c, 16) * ECH

            def e_body(e, _):
                ld = sldst[pl.ds(r0 + e, 16)][0]
                for j in range(NJ):
                    dsj = pl.ds(j * 16, 16)
                    m = a_buf[buf, e, dsj] + b_v[ld, dsj]
                    m = jnp.maximum(m, LEAK * m)
                    acc[ld, dsj] = jnp.maximum(acc[ld, dsj], m)
                return 0
            lax.fori_loop(0, ECH, e_body, 0, unroll=2)
            return 0
        lax.fori_loop(0, nch, ch_body, 0)

    zerosf = jnp.zeros((16,), jnp.float32)
    NWB = NPW // 32

    def h_load(rc):
        pltpu.async_copy(h_hbm.at[pl.ds(w * NPW + rc * 32, 32)],
                         hstage.at[rc % 2], sem_wl)

    def h_load_wait(rc):
        pltpu.make_async_copy(h_hbm.at[pl.ds(w * NPW + rc * 32, 32)],
                              hstage.at[rc % 2], sem_wl).wait()

    def h_store(rc):
        pltpu.async_copy(hstage.at[rc % 2],
                         hnew_hbm.at[pl.ds(w * NPW + rc * 32, 32)], sem_ws)

    def h_store_wait(rc):
        pltpu.make_async_copy(hstage.at[rc % 2],
                              hnew_hbm.at[pl.ds(w * NPW + rc * 32, 32)],
                              sem_ws).wait()

    h_load(0)
    for rc in range(NWB):
        if rc >= 2:
            h_store_wait(rc - 2)
        h_load_wait(rc)
        if rc + 1 < NWB:
            h_load(rc + 1)

        def wb_body(r, _, rc=rc):
            for j in range(NJ):
                dsj = pl.ds(j * 16, 16)
                av = acc[rc * 32 + r, dsj]
                fin = jnp.where(av == -jnp.inf, zerosf, av)
                hstage[rc % 2, r, dsj] = hstage[rc % 2, r, dsj] + fin
            return 0
        lax.fori_loop(0, 32, wb_body, 0)
        h_store(rc)
    h_store_wait(NWB - 2)
    h_store_wait(NWB - 1)


def kernel(x, edge_index, W_in, b_in, W_blocks, b_blocks, W_out, b_out):
    src = edge_index[0].astype(jnp.int32)
    dst = edge_index[1].astype(jnp.int32)
    pad = jnp.zeros((1024,), jnp.int32)
    src_p = jnp.concatenate([src, pad])
    dst_p = jnp.concatenate([dst, pad])
    x_p = jnp.zeros((NPAD, x.shape[1]), jnp.float32).at[:N_NODES].set(x)

    h = _matmul(x_p, W_in, b_in, act=True)
    counts = _count_kernel(dst_p)
    bsrc, bldst = _scatter_kernel(src_p, dst_p, counts)

    n_block = W_blocks.shape[0]
    for i in range(n_block):
        Wt = W_blocks[i, :HID, :]
        Wb = W_blocks[i, HID:, :]
        Wcat = jnp.concatenate([Wb, Wt - Wb], axis=1)
        bcat = jnp.concatenate([jnp.zeros_like(b_blocks[i]), b_blocks[i]])
        A, B = _matmul2(h, Wcat, bcat)
        h = _edge_kernel(A, B, h, bsrc, bldst, counts)

    out = _matmul(h, W_out, b_out, act=False)
    return out[:N_NODES]
